# Initial kernel scaffold; baseline (speedup 1.0000x reference)
import jax
import jax.numpy as jnp
from jax.experimental import pallas as pl


def kernel(ae_q, ae_kv, pe_q, pe_kv, Wq, Wk):
    raise NotImplementedError("write your pallas kernel here")


# speedup vs baseline: 8.4146x; 8.4146x over previous
import math

import jax
import jax.numpy as jnp
from jax.experimental import pallas as pl
from jax.experimental.pallas import tpu as pltpu

B = 4096
KNOW = 4096
D_MODEL = 1024
TOP_K = 64
INV_SQRT_D = 1.0 / math.sqrt(D_MODEL)

ROW_BLK = 256


def _proj_kernel(ae_ref, pe_ref, w_ref, out_ref):
    r = 0.5 * (ae_ref[...] + pe_ref[...])
    out_ref[...] = (
        jax.lax.dot_general(
            r, w_ref[...], (((1,), (1,)), ((), ())),
            preferred_element_type=jnp.float32,
        )
        + r
    )


def _f32_sort_key(x):
    u = jax.lax.bitcast_convert_type(x, jnp.uint32)
    neg = (u >> 31) == 1
    return jnp.where(neg, ~u, u | jnp.uint32(0x80000000))


def _key_to_f32(k):
    neg = (k >> 31) == 0
    u = jnp.where(neg, ~k, k & jnp.uint32(0x7FFFFFFF))
    return jax.lax.bitcast_convert_type(u, jnp.float32)


def _sims_kth_kernel(pq_ref, pkv_ref, sims_ref, kth_ref):
    sims = (
        jax.lax.dot_general(
            pq_ref[...], pkv_ref[...], (((1,), (1,)), ((), ())),
            preferred_element_type=jnp.float32,
        )
        * INV_SQRT_D
    )
    sims_ref[...] = sims
    key = _f32_sort_key(sims)
    prefix = jnp.zeros((ROW_BLK, 1), dtype=jnp.uint32)
    for b in range(31, -1, -1):
        t = prefix | jnp.uint32(1 << b)
        cnt = jnp.sum((key >= t).astype(jnp.int32), axis=1, keepdims=True)
        prefix = jnp.where(cnt >= TOP_K, t, prefix)
    kth = _key_to_f32(prefix)
    kth_ref[...] = jnp.broadcast_to(kth, (ROW_BLK, 128))


def _att_kernel(q_ref, k_ref, sims_ref, kth_ref, out_ref):
    att = (
        jax.lax.dot_general(
            q_ref[...], k_ref[...], (((1,), (1,)), ((), ())),
            preferred_element_type=jnp.float32,
        )
        * INV_SQRT_D
    )
    kth = kth_ref[:, 0:1]
    out_ref[...] = jnp.where(sims_ref[...] < kth, jnp.float32(0.0), att)


def kernel(ae_q, ae_kv, pe_q, pe_kv, Wq, Wk):
    n_row = B // ROW_BLK

    proj = pl.pallas_call(
        _proj_kernel,
        grid=(n_row,),
        in_specs=[
            pl.BlockSpec((ROW_BLK, D_MODEL), lambda i: (i, 0)),
            pl.BlockSpec((ROW_BLK, D_MODEL), lambda i: (i, 0)),
            pl.BlockSpec((D_MODEL, D_MODEL), lambda i: (0, 0)),
        ],
        out_specs=pl.BlockSpec((ROW_BLK, D_MODEL), lambda i: (i, 0)),
        out_shape=jax.ShapeDtypeStruct((B, D_MODEL), jnp.float32),
        compiler_params=pltpu.CompilerParams(
            dimension_semantics=("arbitrary",),
        ),
    )
    query = proj(ae_q, pe_q, Wq)
    key_mat = proj(ae_kv, pe_kv, Wk)

    sims, kth = pl.pallas_call(
        _sims_kth_kernel,
        grid=(n_row,),
        in_specs=[
            pl.BlockSpec((ROW_BLK, D_MODEL), lambda i: (i, 0)),
            pl.BlockSpec((KNOW, D_MODEL), lambda i: (0, 0)),
        ],
        out_specs=[
            pl.BlockSpec((ROW_BLK, KNOW), lambda i: (i, 0)),
            pl.BlockSpec((ROW_BLK, 128), lambda i: (i, 0)),
        ],
        out_shape=[
            jax.ShapeDtypeStruct((B, KNOW), jnp.float32),
            jax.ShapeDtypeStruct((B, 128), jnp.float32),
        ],
        compiler_params=pltpu.CompilerParams(
            dimension_semantics=("arbitrary",),
        ),
    )(pe_q, pe_kv)

    COL_BLK = 1024
    n_col = KNOW // COL_BLK
    out = pl.pallas_call(
        _att_kernel,
        grid=(n_row, n_col),
        in_specs=[
            pl.BlockSpec((ROW_BLK, D_MODEL), lambda i, j: (i, 0)),
            pl.BlockSpec((COL_BLK, D_MODEL), lambda i, j: (j, 0)),
            pl.BlockSpec((ROW_BLK, COL_BLK), lambda i, j: (i, j)),
            pl.BlockSpec((ROW_BLK, 128), lambda i, j: (i, 0)),
        ],
        out_specs=pl.BlockSpec((ROW_BLK, COL_BLK), lambda i, j: (i, j)),
        out_shape=jax.ShapeDtypeStruct((B, KNOW), jnp.float32),
        compiler_params=pltpu.CompilerParams(
            dimension_semantics=("arbitrary", "arbitrary"),
        ),
    )(query, key_mat, sims, kth)
    return out


# speedup vs baseline: 8.6341x; 1.0261x over previous
import math

import jax
import jax.numpy as jnp
from jax.experimental import pallas as pl
from jax.experimental.pallas import tpu as pltpu

B = 4096
KNOW = 4096
D_MODEL = 1024
TOP_K = 64
INV_SQRT_D = 1.0 / math.sqrt(D_MODEL)

ROW_BLK = 128


def _proj_kernel(ae_ref, pe_ref, w_ref, out_ref):
    r = 0.5 * (ae_ref[...] + pe_ref[...])
    out_ref[...] = (
        jax.lax.dot_general(
            r, w_ref[...], (((1,), (1,)), ((), ())),
            preferred_element_type=jnp.float32,
        )
        + r
    )


def _f32_sort_key(x):
    u = jax.lax.bitcast_convert_type(x, jnp.uint32)
    neg = (u >> 31) == 1
    return jnp.where(neg, ~u, u | jnp.uint32(0x80000000))


def _key_to_f32(k):
    neg = (k >> 31) == 0
    u = jnp.where(neg, ~k, k & jnp.uint32(0x7FFFFFFF))
    return jax.lax.bitcast_convert_type(u, jnp.float32)


def _mega_kernel(aeq_ref, peq_ref, wq_ref, pkv_ref, key_ref, out_ref):
    sims = (
        jax.lax.dot_general(
            peq_ref[...], pkv_ref[...], (((1,), (1,)), ((), ())),
            preferred_element_type=jnp.float32,
        )
        * INV_SQRT_D
    )
    r = 0.5 * (aeq_ref[...] + peq_ref[...])
    query = (
        jax.lax.dot_general(
            r, wq_ref[...], (((1,), (1,)), ((), ())),
            preferred_element_type=jnp.float32,
        )
        + r
    )
    att = (
        jax.lax.dot_general(
            query, key_ref[...], (((1,), (1,)), ((), ())),
            preferred_element_type=jnp.float32,
        )
        * INV_SQRT_D
    )
    skey = _f32_sort_key(sims)
    prefix = jnp.zeros((ROW_BLK, 1), dtype=jnp.uint32)
    for b in range(31, -1, -1):
        t = prefix | jnp.uint32(1 << b)
        cnt = jnp.sum((skey >= t).astype(jnp.int32), axis=1, keepdims=True)
        prefix = jnp.where(cnt >= TOP_K, t, prefix)
    kth = _key_to_f32(prefix)
    out_ref[...] = jnp.where(sims < kth, jnp.float32(0.0), att)


def kernel(ae_q, ae_kv, pe_q, pe_kv, Wq, Wk):
    n_row = B // ROW_BLK

    key_mat = pl.pallas_call(
        _proj_kernel,
        grid=(n_row,),
        in_specs=[
            pl.BlockSpec((ROW_BLK, D_MODEL), lambda i: (i, 0)),
            pl.BlockSpec((ROW_BLK, D_MODEL), lambda i: (i, 0)),
            pl.BlockSpec((D_MODEL, D_MODEL), lambda i: (0, 0)),
        ],
        out_specs=pl.BlockSpec((ROW_BLK, D_MODEL), lambda i: (i, 0)),
        out_shape=jax.ShapeDtypeStruct((KNOW, D_MODEL), jnp.float32),
        compiler_params=pltpu.CompilerParams(
            dimension_semantics=("arbitrary",),
        ),
    )(ae_kv, pe_kv, Wk)

    out = pl.pallas_call(
        _mega_kernel,
        grid=(n_row,),
        in_specs=[
            pl.BlockSpec((ROW_BLK, D_MODEL), lambda i: (i, 0)),
            pl.BlockSpec((ROW_BLK, D_MODEL), lambda i: (i, 0)),
            pl.BlockSpec((D_MODEL, D_MODEL), lambda i: (0, 0)),
            pl.BlockSpec((KNOW, D_MODEL), lambda i: (0, 0)),
            pl.BlockSpec((KNOW, D_MODEL), lambda i: (0, 0)),
        ],
        out_specs=pl.BlockSpec((ROW_BLK, KNOW), lambda i: (i, 0)),
        out_shape=jax.ShapeDtypeStruct((B, KNOW), jnp.float32),
        compiler_params=pltpu.CompilerParams(
            dimension_semantics=("arbitrary",),
        ),
    )(ae_q, pe_q, Wq, pe_kv, key_mat)
    return out
